# baseline (speedup 1.0000x reference)
import functools

import jax
import jax.numpy as jnp
from jax import lax
from jax.experimental import pallas as pl
from jax.experimental.pallas import tpu as pltpu
from jax.experimental.pallas import tpu_sc as plsc

_NC = 2
_NS = 16
_NW = _NC * _NS
_L = 16

_BLK = 25600


def _make_proj_body(woff, d_i):
    def _proj_body(itt_ref, fcw_ref, pi_ref):
        wi = fcw_ref[:, woff:woff + d_i]
        pi_ref[...] = jnp.dot(wi, itt_ref[...],
                              preferred_element_type=jnp.float32)[0]
    return _proj_body


def _project_item_table(it_t, fcw, woff):
    d, v = it_t.shape
    grid = (v + _BLK - 1) // _BLK
    return pl.pallas_call(
        _make_proj_body(woff, d),
        grid=(grid,),
        in_specs=[
            pl.BlockSpec((d, _BLK), lambda j: (0, j)),
            pl.BlockSpec(fcw.shape, lambda j: (0, 0)),
        ],
        out_specs=pl.BlockSpec((_BLK,), lambda j: (j,)),
        out_shape=jax.ShapeDtypeStruct((v,), jnp.float32),
    )(it_t, fcw)


def _make_pf_body(woff, d_f):
    def _pf_body(ftt_ref, fcw_ref, pf_ref):
        wf = fcw_ref[:, woff:woff + d_f]
        pf_ref[...] = jnp.dot(wf, ftt_ref[...],
                              preferred_element_type=jnp.float32)[0]
    return _pf_body


def _project_feature_table(ft_t, fcw, woff):
    d, v_f = ft_t.shape
    return pl.pallas_call(
        _make_pf_body(woff, d),
        out_shape=jax.ShapeDtypeStruct((v_f,), jnp.float32),
    )(ft_t, fcw)


def _mesh():
    return plsc.VectorSubcoreMesh(
        core_axis_name="c", subcore_axis_name="s",
        num_cores=_NC, num_subcores=_NS)


def _make_sc1(b, f_len, d_u, d_i, d_s, v_f, v_u):
    bpw = b // _NW
    n_grp = bpw // _L
    w_stage = d_u + d_i + d_s

    @functools.partial(
        pl.kernel,
        out_type=jax.ShapeDtypeStruct((b,), jnp.float32),
        mesh=_mesh(),
        compiler_params=pltpu.CompilerParams(
            needs_layout_passes=False, use_tc_tiling_on_sc=False),
        scratch_types=[
            pltpu.VMEM((bpw,), jnp.int32),
            pltpu.VMEM((bpw,), jnp.int32),
            pltpu.VMEM((f_len, bpw), jnp.int32),
            pltpu.VMEM((bpw,), jnp.int32),
            pltpu.VMEM((bpw * d_u,), jnp.int32),
            pltpu.VMEM((bpw * d_u,), jnp.float32),
            pltpu.VMEM((bpw, d_s), jnp.float32),
            pltpu.VMEM((v_f,), jnp.float32),
            pltpu.VMEM((w_stage + _L,), jnp.float32),
            pltpu.VMEM((bpw,), jnp.float32),
            [pltpu.SemaphoreType.DMA] * 6,
            [pltpu.SemaphoreType.DMA] * 4,
            pltpu.SemaphoreType.DMA,
        ],
    )
    def sc1(uidx, sidx, fidx_t, flen, ut_flat, st, pf, ws, out,
            uidx_v, sidx_v, fidx_v, flen_v, gidx_v, uval_v, srows_v,
            pf_v, ws_v, out_v, sems, sems_s, sem_u):
        wid = lax.axis_index("s") * _NC + lax.axis_index("c")
        base = wid * bpw

        c_stage = [
            pltpu.async_copy(sidx.at[pl.ds(base, bpw)], sidx_v, sems[0]),
            pltpu.async_copy(uidx.at[pl.ds(base, bpw)], uidx_v, sems[1]),
            pltpu.async_copy(fidx_t.at[:, pl.ds(base, bpw)], fidx_v, sems[2]),
            pltpu.async_copy(flen.at[pl.ds(base, bpw)], flen_v, sems[3]),
            pltpu.async_copy(pf, pf_v, sems[4]),
            pltpu.async_copy(ws.at[pl.ds(0, w_stage)],
                             ws_v.at[pl.ds(0, w_stage)], sems[5]),
        ]
        c_stage[0].wait()
        n_sp = len(sems_s)
        spr = bpw // n_sp
        cs = [
            pltpu.async_copy(st.at[sidx_v.at[pl.ds(k * spr, spr)]],
                             srows_v.at[pl.ds(k * spr, spr)], sems_s[k])
            for k in range(n_sp)]

        lanes = lax.iota(jnp.int32, _L)
        ridx = [g * _L + lanes for g in range(n_grp)]

        c_stage[1].wait()
        ug = [uidx_v[pl.ds(g * _L, _L)] for g in range(n_grp)]
        addr0 = [r * d_u for r in ridx]

        def bbody(_, carry):
            dv = carry[0]
            addrs = carry[1:]
            for g in range(n_grp):
                plsc.store_scatter(gidx_v, [addrs[g]], ug[g] + dv)
            return (dv + v_u, *[a + 1 for a in addrs])

        lax.fori_loop(0, d_u, bbody,
                      (jnp.zeros((_L,), jnp.int32), *addr0), unroll=4)
        cu = pltpu.async_copy(ut_flat.at[gidx_v], uval_v, sem_u)

        c_stage[2].wait()
        c_stage[3].wait()
        c_stage[4].wait()
        for g in range(n_grp):
            facc = jnp.zeros((_L,), jnp.float32)
            lv = flen_v[pl.ds(g * _L, _L)].astype(jnp.float32)
            for f in range(f_len):
                idxv = fidx_v[f, pl.ds(g * _L, _L)]
                vals = plsc.load_gather(pf_v, [idxv])
                facc = facc + jnp.where(jnp.float32(f) < lv, vals, 0.0)
            out_v[pl.ds(g * _L, _L)] = facc / lv

        c_stage[5].wait()
        accs = tuple(out_v[pl.ds(g * _L, _L)] for g in range(n_grp))

        for c in cs:
            c.wait()

        def sbody(_, carry):
            col, wcol, *accs = carry
            wv = plsc.load_gather(ws_v, [wcol])
            accs = [
                a + plsc.load_gather(srows_v, [ridx[g], col]) * wv
                for g, a in enumerate(accs)]
            return (col + 1, wcol + 1, *accs)

        carry = lax.fori_loop(
            0, d_s, sbody,
            (jnp.zeros((_L,), jnp.int32),
             jnp.full((_L,), d_u + d_i, jnp.int32), *accs), unroll=4)
        accs = carry[2:]

        cu.wait()

        def ubody(_, carry):
            wcol = carry[0]
            addrs = carry[1:1 + n_grp]
            accs = carry[1 + n_grp:]
            wv = plsc.load_gather(ws_v, [wcol])
            accs = [
                a + plsc.load_gather(uval_v, [addrs[g]]) * wv
                for g, a in enumerate(accs)]
            return (wcol + 1, *[a + 1 for a in addrs], *accs)

        carry = lax.fori_loop(
            0, d_u, ubody,
            (jnp.zeros((_L,), jnp.int32), *addr0, *accs), unroll=4)
        carry = (None, *carry[1 + n_grp:])
        for g in range(n_grp):
            out_v[pl.ds(g * _L, _L)] = carry[1 + g]

        pltpu.sync_copy(out_v, out.at[pl.ds(base, bpw)])

    return sc1


def _make_sc2(b):
    bpw = b // _NW
    n_grp = bpw // _L

    @functools.partial(
        pl.kernel,
        out_type=jax.ShapeDtypeStruct((b,), jnp.float32),
        mesh=_mesh(),
        compiler_params=pltpu.CompilerParams(
            needs_layout_passes=False, use_tc_tiling_on_sc=False),
        scratch_types=[
            pltpu.VMEM((bpw,), jnp.int32),
            pltpu.VMEM((bpw,), jnp.float32),
            pltpu.VMEM((bpw,), jnp.float32),
            pltpu.VMEM((_L,), jnp.float32),
            [pltpu.SemaphoreType.DMA] * 4,
        ],
    )
    def sc2(iidx, part, pi, bias, out,
            iidx_v, part_v, ival_v, bias_v, sems):
        wid = lax.axis_index("s") * _NC + lax.axis_index("c")
        base = wid * bpw

        ci0 = pltpu.async_copy(iidx.at[pl.ds(base, bpw)], iidx_v, sems[0])
        cp = pltpu.async_copy(part.at[pl.ds(base, bpw)], part_v, sems[1])
        cb = pltpu.async_copy(bias.at[pl.ds(0, 1)], bias_v.at[pl.ds(0, 1)],
                              sems[3])
        ci0.wait()
        ci = pltpu.async_copy(pi.at[iidx_v], ival_v, sems[2])
        cp.wait()
        ci.wait()
        cb.wait()
        bvec = plsc.load_gather(bias_v, [jnp.zeros((_L,), jnp.int32)])
        for g in range(n_grp):
            sl = pl.ds(g * _L, _L)
            part_v[sl] = part_v[sl] + ival_v[sl] + bvec
        pltpu.sync_copy(part_v, out.at[pl.ds(base, bpw)])

    return sc2


def kernel(user, item, sentence, feature, feature_len, user_table, item_table,
           feature_table, sent_table, fc_w, fc_b):
    b = user.shape[0]
    f_len = feature.shape[1]
    d_u = user_table.shape[1]
    d_i = item_table.shape[1]
    d_s = sent_table.shape[1]
    v_f = feature_table.shape[0]

    v_u = user_table.shape[0]
    fcw = fc_w.astype(jnp.float32)
    fcw_flat = fcw.reshape(-1)

    pf = _project_feature_table(feature_table.T, fcw, d_u + d_i + d_s)
    pi = _project_item_table(item_table.T, fcw, d_u)
    ut_flat = user_table.T.reshape(-1)

    fidx_t = feature.astype(jnp.int32).T
    flen_i = feature_len.astype(jnp.int32)

    sc1 = _make_sc1(b, f_len, d_u, d_i, d_s, v_f, v_u)
    part = sc1(user.astype(jnp.int32), sentence.astype(jnp.int32),
               fidx_t, flen_i, ut_flat,
               sent_table.astype(jnp.float32), pf, fcw_flat)
    sc2 = _make_sc2(b)
    res = sc2(item.astype(jnp.int32), part, pi, fc_b.astype(jnp.float32))
    return res.reshape(b, 1)


# speedup vs baseline: 1.9478x; 1.9478x over previous
import functools

import jax
import jax.numpy as jnp
from jax import lax
from jax.experimental import pallas as pl
from jax.experimental.pallas import tpu as pltpu
from jax.experimental.pallas import tpu_sc as plsc

_NC = 2
_NS = 16
_NW = _NC * _NS
_L = 16

_BLK = 25600


def _make_proj_body(d_u, d_i):
    def _proj_body(utt_ref, itt_ref, fcw_ref, pu_ref, pi_ref):
        wu = fcw_ref[:, :d_u]
        wi = fcw_ref[:, d_u:d_u + d_i]
        pu_ref[...] = jnp.dot(wu, utt_ref[...],
                              preferred_element_type=jnp.float32)[0]
        pi_ref[...] = jnp.dot(wi, itt_ref[...],
                              preferred_element_type=jnp.float32)[0]
    return _proj_body


def _project_tables(ut_t, it_t, fcw):
    d, v = ut_t.shape
    grid = (v + _BLK - 1) // _BLK
    return pl.pallas_call(
        _make_proj_body(ut_t.shape[0], it_t.shape[0]),
        grid=(grid,),
        in_specs=[
            pl.BlockSpec((d, _BLK), lambda j: (0, j)),
            pl.BlockSpec((d, _BLK), lambda j: (0, j)),
            pl.BlockSpec(fcw.shape, lambda j: (0, 0)),
        ],
        out_specs=[
            pl.BlockSpec((_BLK,), lambda j: (j,)),
            pl.BlockSpec((_BLK,), lambda j: (j,)),
        ],
        out_shape=[
            jax.ShapeDtypeStruct((v,), jnp.float32),
            jax.ShapeDtypeStruct((v,), jnp.float32),
        ],
    )(ut_t, it_t, fcw)


def _make_pf_body(woff, d_f):
    def _pf_body(ftt_ref, fcw_ref, pf_ref):
        wf = fcw_ref[:, woff:woff + d_f]
        pf_ref[...] = jnp.dot(wf, ftt_ref[...],
                              preferred_element_type=jnp.float32)[0]
    return _pf_body


def _project_feature_table(ft_t, fcw, woff):
    d, v_f = ft_t.shape
    return pl.pallas_call(
        _make_pf_body(woff, d),
        out_shape=jax.ShapeDtypeStruct((v_f,), jnp.float32),
    )(ft_t, fcw)


def _mesh():
    return plsc.VectorSubcoreMesh(
        core_axis_name="c", subcore_axis_name="s",
        num_cores=_NC, num_subcores=_NS)


def _make_sc1(b, f_len, d_s, v_f, ws_off):
    bpw = b // _NW
    n_grp = bpw // _L

    @functools.partial(
        pl.kernel,
        out_type=jax.ShapeDtypeStruct((b,), jnp.float32),
        mesh=_mesh(),
        compiler_params=pltpu.CompilerParams(
            needs_layout_passes=False, use_tc_tiling_on_sc=False),
        scratch_types=[
            pltpu.VMEM((bpw,), jnp.int32),
            pltpu.VMEM((f_len, bpw), jnp.int32),
            pltpu.VMEM((bpw,), jnp.int32),
            pltpu.VMEM((bpw, d_s), jnp.float32),
            pltpu.VMEM((v_f,), jnp.float32),
            pltpu.VMEM((d_s + _L,), jnp.float32),
            pltpu.VMEM((bpw,), jnp.float32),
            [pltpu.SemaphoreType.DMA] * 5,
            [pltpu.SemaphoreType.DMA] * 4,
        ],
    )
    def sc1(sidx, fidx_t, flen, st, pf, ws, out,
            sidx_v, fidx_v, flen_v, srows_v, pf_v, ws_v, out_v,
            sems, sems_s):
        wid = lax.axis_index("s") * _NC + lax.axis_index("c")
        base = wid * bpw

        c_stage = [
            pltpu.async_copy(sidx.at[pl.ds(base, bpw)], sidx_v, sems[0]),
            pltpu.async_copy(fidx_t.at[:, pl.ds(base, bpw)], fidx_v, sems[1]),
            pltpu.async_copy(flen.at[pl.ds(base, bpw)], flen_v, sems[2]),
            pltpu.async_copy(pf, pf_v, sems[3]),
            pltpu.async_copy(ws.at[pl.ds(ws_off, d_s)],
                             ws_v.at[pl.ds(0, d_s)], sems[4]),
        ]
        c_stage[0].wait()
        n_sp = len(sems_s)
        spr = bpw // n_sp
        cs = [
            pltpu.async_copy(st.at[sidx_v.at[pl.ds(k * spr, spr)]],
                             srows_v.at[pl.ds(k * spr, spr)], sems_s[k])
            for k in range(n_sp)]
        for c in c_stage[1:]:
            c.wait()

        for g in range(n_grp):
            facc = jnp.zeros((_L,), jnp.float32)
            lv = flen_v[pl.ds(g * _L, _L)].astype(jnp.float32)
            for f in range(f_len):
                idxv = fidx_v[f, pl.ds(g * _L, _L)]
                vals = plsc.load_gather(pf_v, [idxv])
                facc = facc + jnp.where(jnp.float32(f) < lv, vals, 0.0)
            out_v[pl.ds(g * _L, _L)] = facc / lv

        for c in cs:
            c.wait()

        lanes = lax.iota(jnp.int32, _L)
        ridx = [g * _L + lanes for g in range(n_grp)]
        accs = tuple(out_v[pl.ds(g * _L, _L)] for g in range(n_grp))

        def body(_, carry):
            col, *accs = carry
            wv = plsc.load_gather(ws_v, [col])
            accs = [
                a + plsc.load_gather(srows_v, [ridx[g], col]) * wv
                for g, a in enumerate(accs)]
            return (col + 1, *accs)

        carry = lax.fori_loop(
            0, d_s, body, (jnp.zeros((_L,), jnp.int32), *accs), unroll=4)
        for g in range(n_grp):
            out_v[pl.ds(g * _L, _L)] = carry[1 + g]

        pltpu.sync_copy(out_v, out.at[pl.ds(base, bpw)])

    return sc1


def _make_sc2(b):
    bpw = b // _NW
    n_grp = bpw // _L

    @functools.partial(
        pl.kernel,
        out_type=jax.ShapeDtypeStruct((b,), jnp.float32),
        mesh=_mesh(),
        compiler_params=pltpu.CompilerParams(
            needs_layout_passes=False, use_tc_tiling_on_sc=False),
        scratch_types=[
            pltpu.VMEM((bpw,), jnp.int32),
            pltpu.VMEM((bpw,), jnp.int32),
            pltpu.VMEM((bpw,), jnp.float32),
            pltpu.VMEM((bpw,), jnp.float32),
            pltpu.VMEM((bpw,), jnp.float32),
            pltpu.VMEM((_L,), jnp.float32),
            [pltpu.SemaphoreType.DMA] * 6,
        ],
    )
    def sc2(uidx, iidx, part, pu, pi, bias, out,
            uidx_v, iidx_v, part_v, uval_v, ival_v, bias_v, sems):
        wid = lax.axis_index("s") * _NC + lax.axis_index("c")
        base = wid * bpw

        cu0 = pltpu.async_copy(uidx.at[pl.ds(base, bpw)], uidx_v, sems[0])
        ci0 = pltpu.async_copy(iidx.at[pl.ds(base, bpw)], iidx_v, sems[1])
        cp = pltpu.async_copy(part.at[pl.ds(base, bpw)], part_v, sems[2])
        cb = pltpu.async_copy(bias.at[pl.ds(0, 1)], bias_v.at[pl.ds(0, 1)],
                              sems[5])
        cu0.wait()
        cu = pltpu.async_copy(pu.at[uidx_v], uval_v, sems[3])
        ci0.wait()
        ci = pltpu.async_copy(pi.at[iidx_v], ival_v, sems[4])
        cp.wait()
        cu.wait()
        ci.wait()
        cb.wait()
        bvec = plsc.load_gather(bias_v, [jnp.zeros((_L,), jnp.int32)])
        for g in range(n_grp):
            sl = pl.ds(g * _L, _L)
            part_v[sl] = part_v[sl] + uval_v[sl] + ival_v[sl] + bvec
        pltpu.sync_copy(part_v, out.at[pl.ds(base, bpw)])

    return sc2


def kernel(user, item, sentence, feature, feature_len, user_table, item_table,
           feature_table, sent_table, fc_w, fc_b):
    b = user.shape[0]
    f_len = feature.shape[1]
    d_u = user_table.shape[1]
    d_i = item_table.shape[1]
    d_s = sent_table.shape[1]
    v_f = feature_table.shape[0]

    fcw = fc_w.astype(jnp.float32)
    fcw_flat = fcw.reshape(-1)

    pf = _project_feature_table(feature_table.T, fcw, d_u + d_i + d_s)
    pu, pi = _project_tables(user_table.T, item_table.T, fcw)

    fidx_t = feature.astype(jnp.int32).T
    flen_i = feature_len.astype(jnp.int32)

    sc1 = _make_sc1(b, f_len, d_s, v_f, d_u + d_i)
    part = sc1(sentence.astype(jnp.int32), fidx_t, flen_i,
               sent_table.astype(jnp.float32), pf, fcw_flat)
    sc2 = _make_sc2(b)
    res = sc2(user.astype(jnp.int32), item.astype(jnp.int32), part, pu, pi,
              fc_b.astype(jnp.float32))
    return res.reshape(b, 1)


# speedup vs baseline: 1.9538x; 1.0031x over previous
import functools

import jax
import jax.numpy as jnp
from jax import lax
from jax.experimental import pallas as pl
from jax.experimental.pallas import tpu as pltpu
from jax.experimental.pallas import tpu_sc as plsc

_NC = 2
_NS = 16
_NW = _NC * _NS
_L = 16

_BLK = 25600


def _make_proj_body(d_u, d_i):
    def _proj_body(utt_ref, itt_ref, fcw_ref, pu_ref, pi_ref):
        wu = fcw_ref[:, :d_u]
        wi = fcw_ref[:, d_u:d_u + d_i]
        pu_ref[...] = jnp.dot(wu, utt_ref[...],
                              preferred_element_type=jnp.float32)[0]
        pi_ref[...] = jnp.dot(wi, itt_ref[...],
                              preferred_element_type=jnp.float32)[0]
    return _proj_body


def _project_tables(ut_t, it_t, fcw):
    d, v = ut_t.shape
    grid = (v + _BLK - 1) // _BLK
    return pl.pallas_call(
        _make_proj_body(ut_t.shape[0], it_t.shape[0]),
        grid=(grid,),
        in_specs=[
            pl.BlockSpec((d, _BLK), lambda j: (0, j)),
            pl.BlockSpec((d, _BLK), lambda j: (0, j)),
            pl.BlockSpec(fcw.shape, lambda j: (0, 0)),
        ],
        out_specs=[
            pl.BlockSpec((_BLK,), lambda j: (j,)),
            pl.BlockSpec((_BLK,), lambda j: (j,)),
        ],
        out_shape=[
            jax.ShapeDtypeStruct((v,), jnp.float32),
            jax.ShapeDtypeStruct((v,), jnp.float32),
        ],
    )(ut_t, it_t, fcw)


def _make_pf_body(woff, d_f):
    def _pf_body(ftt_ref, fcw_ref, pf_ref):
        wf = fcw_ref[:, woff:woff + d_f]
        pf_ref[...] = jnp.dot(wf, ftt_ref[...],
                              preferred_element_type=jnp.float32)[0]
    return _pf_body


def _project_feature_table(ft_t, fcw, woff):
    d, v_f = ft_t.shape
    return pl.pallas_call(
        _make_pf_body(woff, d),
        out_shape=jax.ShapeDtypeStruct((v_f,), jnp.float32),
    )(ft_t, fcw)


def _mesh():
    return plsc.VectorSubcoreMesh(
        core_axis_name="c", subcore_axis_name="s",
        num_cores=_NC, num_subcores=_NS)


def _make_sc1(b, f_len, d_s, v_f, ws_off):
    bpw = b // _NW
    n_grp = bpw // _L

    @functools.partial(
        pl.kernel,
        out_type=jax.ShapeDtypeStruct((b,), jnp.float32),
        mesh=_mesh(),
        compiler_params=pltpu.CompilerParams(
            needs_layout_passes=False, use_tc_tiling_on_sc=False),
        scratch_types=[
            pltpu.VMEM((bpw,), jnp.int32),
            pltpu.VMEM((f_len, bpw), jnp.int32),
            pltpu.VMEM((bpw,), jnp.int32),
            pltpu.VMEM((bpw, d_s), jnp.float32),
            pltpu.VMEM((v_f,), jnp.float32),
            pltpu.VMEM((d_s + _L,), jnp.float32),
            pltpu.VMEM((bpw,), jnp.float32),
            [pltpu.SemaphoreType.DMA] * 5,
            [pltpu.SemaphoreType.DMA] * 4,
        ],
    )
    def sc1(sidx, fidx_t, flen, st, pf, ws, out,
            sidx_v, fidx_v, flen_v, srows_v, pf_v, ws_v, out_v,
            sems, sems_s):
        wid = lax.axis_index("s") * _NC + lax.axis_index("c")
        base = wid * bpw

        c_stage = [
            pltpu.async_copy(sidx.at[pl.ds(base, bpw)], sidx_v, sems[0]),
            pltpu.async_copy(fidx_t.at[:, pl.ds(base, bpw)], fidx_v, sems[1]),
            pltpu.async_copy(flen.at[pl.ds(base, bpw)], flen_v, sems[2]),
            pltpu.async_copy(pf, pf_v, sems[3]),
            pltpu.async_copy(ws.at[0, pl.ds(ws_off, d_s)],
                             ws_v.at[pl.ds(0, d_s)], sems[4]),
        ]
        c_stage[0].wait()
        n_sp = len(sems_s)
        spr = bpw // n_sp
        cs = [
            pltpu.async_copy(st.at[sidx_v.at[pl.ds(k * spr, spr)]],
                             srows_v.at[pl.ds(k * spr, spr)], sems_s[k])
            for k in range(n_sp)]
        for c in c_stage[1:]:
            c.wait()

        for g in range(n_grp):
            facc = jnp.zeros((_L,), jnp.float32)
            lv = flen_v[pl.ds(g * _L, _L)].astype(jnp.float32)
            for f in range(f_len):
                idxv = fidx_v[f, pl.ds(g * _L, _L)]
                vals = plsc.load_gather(pf_v, [idxv])
                facc = facc + jnp.where(jnp.float32(f) < lv, vals, 0.0)
            out_v[pl.ds(g * _L, _L)] = facc / lv

        for c in cs:
            c.wait()

        lanes = lax.iota(jnp.int32, _L)
        ridx = [g * _L + lanes for g in range(n_grp)]
        accs = tuple(out_v[pl.ds(g * _L, _L)] for g in range(n_grp))

        def body(_, carry):
            col, *accs = carry
            wv = plsc.load_gather(ws_v, [col])
            accs = [
                a + plsc.load_gather(srows_v, [ridx[g], col]) * wv
                for g, a in enumerate(accs)]
            return (col + 1, *accs)

        carry = lax.fori_loop(
            0, d_s, body, (jnp.zeros((_L,), jnp.int32), *accs), unroll=4)
        for g in range(n_grp):
            out_v[pl.ds(g * _L, _L)] = carry[1 + g]

        pltpu.sync_copy(out_v, out.at[pl.ds(base, bpw)])

    return sc1


def _make_sc2(b):
    bpw = b // _NW
    n_grp = bpw // _L

    @functools.partial(
        pl.kernel,
        out_type=jax.ShapeDtypeStruct((b,), jnp.float32),
        mesh=_mesh(),
        compiler_params=pltpu.CompilerParams(
            needs_layout_passes=False, use_tc_tiling_on_sc=False),
        scratch_types=[
            pltpu.VMEM((bpw,), jnp.int32),
            pltpu.VMEM((bpw,), jnp.int32),
            pltpu.VMEM((bpw,), jnp.float32),
            pltpu.VMEM((bpw,), jnp.float32),
            pltpu.VMEM((bpw,), jnp.float32),
            pltpu.VMEM((_L,), jnp.float32),
            [pltpu.SemaphoreType.DMA] * 6,
        ],
    )
    def sc2(uidx, iidx, part, pu, pi, bias, out,
            uidx_v, iidx_v, part_v, uval_v, ival_v, bias_v, sems):
        wid = lax.axis_index("s") * _NC + lax.axis_index("c")
        base = wid * bpw

        cu0 = pltpu.async_copy(uidx.at[pl.ds(base, bpw)], uidx_v, sems[0])
        ci0 = pltpu.async_copy(iidx.at[pl.ds(base, bpw)], iidx_v, sems[1])
        cp = pltpu.async_copy(part.at[pl.ds(base, bpw)], part_v, sems[2])
        cb = pltpu.async_copy(bias.at[pl.ds(0, 1)], bias_v.at[pl.ds(0, 1)],
                              sems[5])
        cu0.wait()
        cu = pltpu.async_copy(pu.at[uidx_v], uval_v, sems[3])
        ci0.wait()
        ci = pltpu.async_copy(pi.at[iidx_v], ival_v, sems[4])
        cp.wait()
        cu.wait()
        ci.wait()
        cb.wait()
        bvec = plsc.load_gather(bias_v, [jnp.zeros((_L,), jnp.int32)])
        for g in range(n_grp):
            sl = pl.ds(g * _L, _L)
            part_v[sl] = part_v[sl] + uval_v[sl] + ival_v[sl] + bvec
        pltpu.sync_copy(part_v, out.at[pl.ds(base, bpw)])

    return sc2


def kernel(user, item, sentence, feature, feature_len, user_table, item_table,
           feature_table, sent_table, fc_w, fc_b):
    b = user.shape[0]
    f_len = feature.shape[1]
    d_u = user_table.shape[1]
    d_i = item_table.shape[1]
    d_s = sent_table.shape[1]
    v_f = feature_table.shape[0]

    fcw = fc_w.astype(jnp.float32)

    pf = _project_feature_table(feature_table.T, fcw, d_u + d_i + d_s)
    pu, pi = _project_tables(user_table.T, item_table.T, fcw)

    fidx_t = feature.astype(jnp.int32).T
    flen_i = feature_len.astype(jnp.int32)

    sc1 = _make_sc1(b, f_len, d_s, v_f, d_u + d_i)
    part = sc1(sentence.astype(jnp.int32), fidx_t, flen_i,
               sent_table.astype(jnp.float32), pf, fcw)
    sc2 = _make_sc2(b)
    res = sc2(user.astype(jnp.int32), item.astype(jnp.int32), part, pu, pi,
              fc_b.astype(jnp.float32))
    return res.reshape(b, 1)
